# baseline (speedup 1.0000x reference)
import functools

import jax
import jax.numpy as jnp
from jax import lax
from jax.experimental import pallas as pl
from jax.experimental.pallas import tpu as pltpu
from jax.experimental.pallas import tpu_sc as plsc

NC = 2
NS = 16
L = 16
NW = NC * NS

BATCH = 16384
CPT = BATCH // NW
G = 128
NG = CPT // G


def _sigmoid(x):
    return 1.0 / (1.0 + jnp.exp(-x))


def _irt_body(diff_hbm, disc_hbm, ab_hbm, idx_hbm, out_hbm,
              idx_v, diff_v, disc_v, ab_v, out_v, sem):
    wid = lax.axis_index("s") * NC + lax.axis_index("c")
    base = wid * CPT

    pltpu.sync_copy(idx_hbm.at[wid], idx_v)

    copies = []
    for i in range(NG):
        copies.append(pltpu.async_copy(diff_hbm.at[idx_v.at[i]], diff_v.at[i], sem))
        copies.append(pltpu.async_copy(disc_hbm.at[idx_v.at[i]], disc_v.at[i], sem))
    copies.append(pltpu.async_copy(ab_hbm.at[pl.ds(base, CPT)], ab_v, sem))
    for c in copies:
        c.wait()

    lane = lax.iota(jnp.int32, L)
    for i in range(NG):
        for k in range(G // L):
            off = k * L
            d = _sigmoid(diff_v[i, pl.ds(off, L)])
            q = _sigmoid(disc_v[i, pl.ds(off, L)])
            z = q * (ab_v[pl.ds(i * G + off, L)] - d) * 1.7
            o1 = 1.0 / (1.0 + jnp.exp(-z))
            o0 = 1.0 / (1.0 + jnp.exp(z))
            pos = (i * G + off) * 2 + lane * 2
            plsc.store_scatter(out_v, [pos], o0)
            plsc.store_scatter(out_v, [pos + 1], o1)

    pltpu.sync_copy(out_v, out_hbm.at[pl.ds(base * 2, CPT * 2)])


@functools.partial(
    pl.kernel,
    out_type=jax.ShapeDtypeStruct((2 * BATCH,), jnp.float32),
    mesh=plsc.VectorSubcoreMesh(
        core_axis_name="c", subcore_axis_name="s",
        num_cores=NC, num_subcores=NS),
    compiler_params=pltpu.CompilerParams(needs_layout_passes=False),
    scratch_types=[
        pltpu.VMEM((NG, G), jnp.int32),
        pltpu.VMEM((NG, G), jnp.float32),
        pltpu.VMEM((NG, G), jnp.float32),
        pltpu.VMEM((CPT,), jnp.float32),
        pltpu.VMEM((2 * CPT,), jnp.float32),
        pltpu.SemaphoreType.DMA,
    ],
)
def _irt_kernel(diff_hbm, disc_hbm, ab_hbm, idx_hbm, out_hbm, *scratch):
    _irt_body(diff_hbm, disc_hbm, ab_hbm, idx_hbm, out_hbm, *scratch)


def kernel(ability, exer_id, e_difficulty, e_discrimination):
    idx = exer_id.astype(jnp.int32).reshape(NW, NG, G)
    out = _irt_kernel(
        e_difficulty.reshape(-1),
        e_discrimination.reshape(-1),
        ability.reshape(-1),
        idx,
    )
    return out.reshape(BATCH, 2)


# speedup vs baseline: 1.1546x; 1.1546x over previous
import functools

import jax
import jax.numpy as jnp
from jax import lax
from jax.experimental import pallas as pl
from jax.experimental.pallas import tpu as pltpu
from jax.experimental.pallas import tpu_sc as plsc

NC = 2
NS = 16
L = 16
NW = NC * NS

BATCH = 16384
CPT = BATCH // NW
G = 128
NG = CPT // G
W = 8


def _irt_body(diff_hbm, disc_hbm, ab_hbm, idx_hbm, out_hbm,
              idx_v, row_v, diff_v, disc_v, ab_v, out_v, sem):
    wid = lax.axis_index("s") * NC + lax.axis_index("c")
    base = wid * CPT

    pltpu.sync_copy(idx_hbm.at[wid], idx_v)

    for i in range(NG):
        for k in range(G // L):
            idx16 = idx_v[i, pl.ds(k * L, L)]
            row_v[i, pl.ds(k * L, L)] = lax.shift_right_logical(idx16, 3)

    copies = []
    for i in range(NG):
        copies.append(pltpu.async_copy(
            diff_hbm.at[row_v.at[i]], diff_v.at[pl.ds(i * G, G)], sem))
        copies.append(pltpu.async_copy(
            disc_hbm.at[row_v.at[i]], disc_v.at[pl.ds(i * G, G)], sem))
    copies.append(pltpu.async_copy(ab_hbm.at[pl.ds(base, CPT)], ab_v, sem))
    for c in copies:
        c.wait()

    lane = lax.iota(jnp.int32, L)
    for i in range(NG):
        for k in range(G // L):
            off = i * G + k * L
            rows = lane + off
            col = jnp.bitwise_and(idx_v[i, pl.ds(k * L, L)], W - 1)
            d_raw = plsc.load_gather(diff_v, [rows, col])
            q_raw = plsc.load_gather(disc_v, [rows, col])
            d = 1.0 / (1.0 + jnp.exp(-d_raw))
            q = 1.0 / (1.0 + jnp.exp(-q_raw))
            z = q * (ab_v[pl.ds(off, L)] - d) * 1.7
            o1 = 1.0 / (1.0 + jnp.exp(-z))
            out_v[pl.ds(i * 2 * G + k * L, L)] = 1.0 - o1
            out_v[pl.ds(i * 2 * G + G + k * L, L)] = o1

    pltpu.sync_copy(out_v, out_hbm.at[pl.ds(base * 2, CPT * 2)])


@functools.partial(
    pl.kernel,
    out_type=jax.ShapeDtypeStruct((2 * BATCH,), jnp.float32),
    mesh=plsc.VectorSubcoreMesh(
        core_axis_name="c", subcore_axis_name="s",
        num_cores=NC, num_subcores=NS),
    compiler_params=pltpu.CompilerParams(
        needs_layout_passes=False, use_tc_tiling_on_sc=False),
    scratch_types=[
        pltpu.VMEM((NG, G), jnp.int32),
        pltpu.VMEM((NG, G), jnp.int32),
        pltpu.VMEM((CPT, W), jnp.float32),
        pltpu.VMEM((CPT, W), jnp.float32),
        pltpu.VMEM((CPT,), jnp.float32),
        pltpu.VMEM((2 * CPT,), jnp.float32),
        pltpu.SemaphoreType.DMA,
    ],
)
def _irt_kernel(diff_hbm, disc_hbm, ab_hbm, idx_hbm, out_hbm, *scratch):
    _irt_body(diff_hbm, disc_hbm, ab_hbm, idx_hbm, out_hbm, *scratch)


def kernel(ability, exer_id, e_difficulty, e_discrimination):
    idx = exer_id.astype(jnp.int32).reshape(NW, NG, G)
    n = e_difficulty.shape[0]
    pad = (-n) % G
    diff_t = jnp.pad(e_difficulty, ((0, pad), (0, 0))).reshape(-1, W)
    disc_t = jnp.pad(e_discrimination, ((0, pad), (0, 0))).reshape(-1, W)
    out = _irt_kernel(
        diff_t,
        disc_t,
        ability.reshape(-1),
        idx,
    )
    return out.reshape(BATCH // G, 2, G).swapaxes(1, 2).reshape(BATCH, 2)


# speedup vs baseline: 3.5728x; 3.0944x over previous
import functools

import jax
import jax.numpy as jnp
from jax import lax
from jax.experimental import pallas as pl
from jax.experimental.pallas import tpu as pltpu
from jax.experimental.pallas import tpu_sc as plsc

NC = 2
NS = 16
L = 16
NW = NC * NS

BATCH = 16384
CPT = BATCH // NW
G = 128
NG = CPT // G
W = 8

EXN = 1000000
HEAD = 999424
TAIL = EXN - HEAD
HEAD_ROWS = HEAD // W


def _irt_body(diff_hbm, disc_hbm, dtail_hbm, qtail_hbm, ab_hbm, idx_hbm,
              out_hbm, idx_v, row_v, diff_v, disc_v, dtail_v, qtail_v,
              ab_v, out_v, sem):
    wid = lax.axis_index("s") * NC + lax.axis_index("c")
    base = wid * CPT

    pltpu.sync_copy(idx_hbm.at[wid], idx_v)

    last = jnp.full((L,), HEAD_ROWS - 1, jnp.int32)
    for i in range(NG):
        for k in range(G // L):
            idx16 = idx_v[i, pl.ds(k * L, L)]
            row_v[i, pl.ds(k * L, L)] = jnp.minimum(
                lax.shift_right_logical(idx16, 3), last)

    copies = []
    for i in range(NG):
        copies.append(pltpu.async_copy(
            diff_hbm.at[row_v.at[i]], diff_v.at[pl.ds(i * G, G)], sem))
        copies.append(pltpu.async_copy(
            disc_hbm.at[row_v.at[i]], disc_v.at[pl.ds(i * G, G)], sem))
    copies.append(pltpu.async_copy(ab_hbm.at[pl.ds(base, CPT)], ab_v, sem))
    copies.append(pltpu.async_copy(dtail_hbm, dtail_v, sem))
    copies.append(pltpu.async_copy(qtail_hbm, qtail_v, sem))
    for c in copies:
        c.wait()

    lane = lax.iota(jnp.int32, L)
    head_n = jnp.full((L,), HEAD, jnp.int32)
    for i in range(NG):
        for k in range(G // L):
            off = i * G + k * L
            rows = lane + off
            iv = idx_v[i, pl.ds(k * L, L)]
            col = jnp.bitwise_and(iv, W - 1)
            in_tail = iv >= head_n
            t_off = jnp.maximum(iv - head_n, 0)
            t_row = lax.shift_right_logical(t_off, 3)
            d_raw = jnp.where(
                in_tail,
                plsc.load_gather(dtail_v, [t_row, col]),
                plsc.load_gather(diff_v, [rows, col]))
            q_raw = jnp.where(
                in_tail,
                plsc.load_gather(qtail_v, [t_row, col]),
                plsc.load_gather(disc_v, [rows, col]))
            d = 1.0 / (1.0 + jnp.exp(-d_raw))
            q = 1.0 / (1.0 + jnp.exp(-q_raw))
            z = q * (ab_v[pl.ds(off, L)] - d) * 1.7
            o1 = 1.0 / (1.0 + jnp.exp(-z))
            out_v[pl.ds(i * 2 * G + k * L, L)] = 1.0 - o1
            out_v[pl.ds(i * 2 * G + G + k * L, L)] = o1

    pltpu.sync_copy(out_v, out_hbm.at[pl.ds(base * 2, CPT * 2)])


@functools.partial(
    pl.kernel,
    out_type=jax.ShapeDtypeStruct((2 * BATCH,), jnp.float32),
    mesh=plsc.VectorSubcoreMesh(
        core_axis_name="c", subcore_axis_name="s",
        num_cores=NC, num_subcores=NS),
    compiler_params=pltpu.CompilerParams(
        needs_layout_passes=False, use_tc_tiling_on_sc=False),
    scratch_types=[
        pltpu.VMEM((NG, G), jnp.int32),
        pltpu.VMEM((NG, G), jnp.int32),
        pltpu.VMEM((CPT, W), jnp.float32),
        pltpu.VMEM((CPT, W), jnp.float32),
        pltpu.VMEM((TAIL // W, W), jnp.float32),
        pltpu.VMEM((TAIL // W, W), jnp.float32),
        pltpu.VMEM((CPT,), jnp.float32),
        pltpu.VMEM((2 * CPT,), jnp.float32),
        pltpu.SemaphoreType.DMA,
    ],
)
def _irt_kernel(diff_hbm, disc_hbm, dtail_hbm, qtail_hbm, ab_hbm, idx_hbm,
                out_hbm, *scratch):
    _irt_body(diff_hbm, disc_hbm, dtail_hbm, qtail_hbm, ab_hbm, idx_hbm,
              out_hbm, *scratch)


def kernel(ability, exer_id, e_difficulty, e_discrimination):
    idx = exer_id.astype(jnp.int32).reshape(NW, NG, G)
    diff_h, disc_h = lax.optimization_barrier((
        lax.slice(e_difficulty, (0, 0), (HEAD, 1)),
        lax.slice(e_discrimination, (0, 0), (HEAD, 1))))
    diff_h = diff_h.reshape(-1, W)
    disc_h = disc_h.reshape(-1, W)
    diff_t = lax.slice(e_difficulty, (HEAD, 0), (EXN, 1)).reshape(-1, W)
    disc_t = lax.slice(e_discrimination, (HEAD, 0), (EXN, 1)).reshape(-1, W)
    out = _irt_kernel(diff_h, disc_h, diff_t, disc_t, ability.reshape(-1), idx)
    return out.reshape(BATCH // G, 2, G).swapaxes(1, 2).reshape(BATCH, 2)


# speedup vs baseline: 3.6374x; 1.0181x over previous
import functools

import jax
import jax.numpy as jnp
from jax import lax
from jax.experimental import pallas as pl
from jax.experimental.pallas import tpu as pltpu
from jax.experimental.pallas import tpu_sc as plsc

NC = 2
NS = 16
L = 16
NW = NC * NS

BATCH = 16384
CPT = BATCH // NW
G = 128
NG = CPT // G
W = 8

EXN = 1000000
HEAD = 999424
TAIL = EXN - HEAD
HEAD_ROWS = HEAD // W


def _irt_body(diff_hbm, disc_hbm, dtail_hbm, qtail_hbm, ab_hbm, idx_hbm,
              out_hbm, idx_v, row_v, diff_v, disc_v, dtail_v, qtail_v,
              ab_v, out_v, sem, gsem):
    wid = lax.axis_index("s") * NC + lax.axis_index("c")
    base = wid * CPT

    pltpu.sync_copy(idx_hbm.at[wid], idx_v)

    small = [
        pltpu.async_copy(ab_hbm.at[pl.ds(base, CPT)], ab_v, sem),
        pltpu.async_copy(dtail_hbm, dtail_v, sem),
        pltpu.async_copy(qtail_hbm, qtail_v, sem),
    ]

    last = jnp.full((L,), HEAD_ROWS - 1, jnp.int32)
    gathers = []
    for i in range(NG):
        for k in range(G // L):
            idx16 = idx_v[i, pl.ds(k * L, L)]
            row_v[i, pl.ds(k * L, L)] = jnp.minimum(
                lax.shift_right_logical(idx16, 3), last)
        gathers.append((
            pltpu.async_copy(
                diff_hbm.at[row_v.at[i]], diff_v.at[pl.ds(i * G, G)],
                gsem.at[i]),
            pltpu.async_copy(
                disc_hbm.at[row_v.at[i]], disc_v.at[pl.ds(i * G, G)],
                gsem.at[i])))
    for c in small:
        c.wait()

    lane = lax.iota(jnp.int32, L)
    head_n = jnp.full((L,), HEAD, jnp.int32)
    for i in range(NG):
        for c in gathers[i]:
            c.wait()
        for k in range(G // L):
            off = i * G + k * L
            rows = lane + off
            iv = idx_v[i, pl.ds(k * L, L)]
            col = jnp.bitwise_and(iv, W - 1)
            in_tail = iv >= head_n
            t_off = jnp.maximum(iv - head_n, 0)
            t_row = lax.shift_right_logical(t_off, 3)
            d_raw = jnp.where(
                in_tail,
                plsc.load_gather(dtail_v, [t_row, col]),
                plsc.load_gather(diff_v, [rows, col]))
            q_raw = jnp.where(
                in_tail,
                plsc.load_gather(qtail_v, [t_row, col]),
                plsc.load_gather(disc_v, [rows, col]))
            d = 1.0 / (1.0 + jnp.exp(-d_raw))
            q = 1.0 / (1.0 + jnp.exp(-q_raw))
            z = q * (ab_v[pl.ds(off, L)] - d) * 1.7
            o1 = 1.0 / (1.0 + jnp.exp(-z))
            out_v[pl.ds(i * 2 * G + k * L, L)] = 1.0 - o1
            out_v[pl.ds(i * 2 * G + G + k * L, L)] = o1

    pltpu.sync_copy(out_v, out_hbm.at[pl.ds(base * 2, CPT * 2)])


@functools.partial(
    pl.kernel,
    out_type=jax.ShapeDtypeStruct((2 * BATCH,), jnp.float32),
    mesh=plsc.VectorSubcoreMesh(
        core_axis_name="c", subcore_axis_name="s",
        num_cores=NC, num_subcores=NS),
    compiler_params=pltpu.CompilerParams(
        needs_layout_passes=False, use_tc_tiling_on_sc=False),
    scratch_types=[
        pltpu.VMEM((NG, G), jnp.int32),
        pltpu.VMEM((NG, G), jnp.int32),
        pltpu.VMEM((CPT, W), jnp.float32),
        pltpu.VMEM((CPT, W), jnp.float32),
        pltpu.VMEM((TAIL // W, W), jnp.float32),
        pltpu.VMEM((TAIL // W, W), jnp.float32),
        pltpu.VMEM((CPT,), jnp.float32),
        pltpu.VMEM((2 * CPT,), jnp.float32),
        pltpu.SemaphoreType.DMA,
        pltpu.SemaphoreType.DMA((NG,)),
    ],
)
def _irt_kernel(diff_hbm, disc_hbm, dtail_hbm, qtail_hbm, ab_hbm, idx_hbm,
                out_hbm, *scratch):
    _irt_body(diff_hbm, disc_hbm, dtail_hbm, qtail_hbm, ab_hbm, idx_hbm,
              out_hbm, *scratch)


def kernel(ability, exer_id, e_difficulty, e_discrimination):
    idx = exer_id.astype(jnp.int32).reshape(NW, NG, G)
    diff_h = lax.slice(e_difficulty, (0, 0), (HEAD, 1)).reshape(-1, W)
    disc_h = lax.slice(e_discrimination, (0, 0), (HEAD, 1)).reshape(-1, W)
    diff_t = lax.slice(e_difficulty, (HEAD, 0), (EXN, 1)).reshape(-1, W)
    disc_t = lax.slice(e_discrimination, (HEAD, 0), (EXN, 1)).reshape(-1, W)
    out = _irt_kernel(diff_h, disc_h, diff_t, disc_t, ability.reshape(-1), idx)
    return out.reshape(BATCH // G, 2, G).swapaxes(1, 2).reshape(BATCH, 2)
